# baseline (speedup 1.0000x reference)
import jax
import jax.numpy as jnp
from jax.experimental import pallas as pl
from jax.experimental.pallas import tpu as pltpu

_N_STATES = 4
_N_ACTIONS = 2
_PACK = 32
_HID = 64
_BLOCK_R = 512


def _fused_mlp_kernel(xr_ref, m1_ref, b1e_ref, m2_ref, b2e_ref, o_ref):
    h = jnp.maximum(
        jnp.dot(xr_ref[...], m1_ref[...], preferred_element_type=jnp.float32)
        + b1e_ref[...],
        0.0,
    )
    o_ref[...] = (
        jnp.dot(h, m2_ref[...], preferred_element_type=jnp.float32)
        + b2e_ref[...]
    )


def kernel(x, w1p, b1p, w2p, b2p):
    B = x.shape[0]
    R = B // _PACK

    xr = jnp.reshape(x, (R, _PACK * _N_STATES))

    eye = jnp.eye(_PACK, dtype=jnp.float32)
    w1t = jnp.transpose(w1p[:_HID, :])
    m1 = jnp.kron(eye, w1t)
    b1e = jnp.tile(jnp.transpose(b1p[:_HID, :]), (1, _PACK))
    w2t = jnp.transpose(w2p[:_N_ACTIONS, :_HID])
    m2 = jnp.kron(eye, w2t)
    b2e = jnp.tile(jnp.transpose(b2p[:_N_ACTIONS, :]), (1, _PACK))

    block_r = min(_BLOCK_R, R)
    num_blocks = R // block_r

    out = pl.pallas_call(
        _fused_mlp_kernel,
        out_shape=jax.ShapeDtypeStruct((R, _PACK * _N_ACTIONS), jnp.float32),
        grid=(num_blocks,),
        in_specs=[
            pl.BlockSpec((block_r, _PACK * _N_STATES), lambda i: (i, 0)),
            pl.BlockSpec((_PACK * _N_STATES, _PACK * _HID), lambda i: (0, 0)),
            pl.BlockSpec((1, _PACK * _HID), lambda i: (0, 0)),
            pl.BlockSpec((_PACK * _HID, _PACK * _N_ACTIONS), lambda i: (0, 0)),
            pl.BlockSpec((1, _PACK * _N_ACTIONS), lambda i: (0, 0)),
        ],
        out_specs=pl.BlockSpec((block_r, _PACK * _N_ACTIONS), lambda i: (i, 0)),
        compiler_params=pltpu.CompilerParams(
            dimension_semantics=("parallel",)),
    )(xr, m1, b1e, m2, b2e)
    return jnp.reshape(out, (B, _N_ACTIONS))


# speedup vs baseline: 31.6624x; 31.6624x over previous
import jax
import jax.numpy as jnp
from jax.experimental import pallas as pl
from jax.experimental.pallas import tpu as pltpu

_N_STATES = 4
_N_ACTIONS = 2
_HID = 64
_BLOCK_L = 16384


def _mlp_t_kernel(xT_ref, w1_ref, b1_ref, w2_ref, b2_ref, oT_ref):
    h = jnp.maximum(
        jnp.dot(w1_ref[...], xT_ref[...], preferred_element_type=jnp.float32)
        + b1_ref[...],
        0.0,
    )
    oT_ref[...] = (
        jnp.dot(w2_ref[...], h, preferred_element_type=jnp.float32)
        + b2_ref[...]
    )


def kernel(x, w1p, b1p, w2p, b2p):
    B = x.shape[0]
    xT = jnp.transpose(x)
    w1s = w1p[:_HID, :]
    b1s = b1p[:_HID, :]
    w2s = w2p[:_N_ACTIONS, :_HID]
    b2s = b2p[:_N_ACTIONS, :]

    block_l = min(_BLOCK_L, B)
    num_blocks = B // block_l

    oT = pl.pallas_call(
        _mlp_t_kernel,
        out_shape=jax.ShapeDtypeStruct((_N_ACTIONS, B), jnp.float32),
        grid=(num_blocks,),
        in_specs=[
            pl.BlockSpec((_N_STATES, block_l), lambda i: (0, i)),
            pl.BlockSpec((_HID, _N_STATES), lambda i: (0, 0)),
            pl.BlockSpec((_HID, 1), lambda i: (0, 0)),
            pl.BlockSpec((_N_ACTIONS, _HID), lambda i: (0, 0)),
            pl.BlockSpec((_N_ACTIONS, 1), lambda i: (0, 0)),
        ],
        out_specs=pl.BlockSpec((_N_ACTIONS, block_l), lambda i: (0, i)),
        compiler_params=pltpu.CompilerParams(
            dimension_semantics=("parallel",)),
    )(xT, w1s, b1s, w2s, b2s)
    return jnp.transpose(oT)


# speedup vs baseline: 35.1193x; 1.1092x over previous
import jax
import jax.numpy as jnp
from jax.experimental import pallas as pl
from jax.experimental.pallas import tpu as pltpu

_N_STATES = 4
_N_ACTIONS = 2
_HID = 64
_BLOCK_L = 32768


def _mlp_t_kernel(xT_ref, w1_ref, b1_ref, w2_ref, b2_ref, oT_ref):
    h = jnp.maximum(
        jnp.dot(w1_ref[...], xT_ref[...], preferred_element_type=jnp.float32)
        + b1_ref[...],
        0.0,
    )
    oT_ref[...] = (
        jnp.dot(w2_ref[...], h, preferred_element_type=jnp.float32)
        + b2_ref[...]
    )


def kernel(x, w1p, b1p, w2p, b2p):
    B = x.shape[0]
    xT = jnp.transpose(x)
    w1s = w1p[:_HID, :]
    b1s = b1p[:_HID, :]
    w2s = w2p[:_N_ACTIONS, :_HID]
    b2s = b2p[:_N_ACTIONS, :]

    block_l = min(_BLOCK_L, B)
    num_blocks = B // block_l

    oT = pl.pallas_call(
        _mlp_t_kernel,
        out_shape=jax.ShapeDtypeStruct((_N_ACTIONS, B), jnp.float32),
        grid=(num_blocks,),
        in_specs=[
            pl.BlockSpec((_N_STATES, block_l), lambda i: (0, i)),
            pl.BlockSpec((_HID, _N_STATES), lambda i: (0, 0)),
            pl.BlockSpec((_HID, 1), lambda i: (0, 0)),
            pl.BlockSpec((_N_ACTIONS, _HID), lambda i: (0, 0)),
            pl.BlockSpec((_N_ACTIONS, 1), lambda i: (0, 0)),
        ],
        out_specs=pl.BlockSpec((_N_ACTIONS, block_l), lambda i: (0, i)),
        compiler_params=pltpu.CompilerParams(
            dimension_semantics=("parallel",)),
    )(xT, w1s, b1s, w2s, b2s)
    return jnp.transpose(oT)


# speedup vs baseline: 36.4636x; 1.0383x over previous
import jax
import jax.numpy as jnp
from jax.experimental import pallas as pl
from jax.experimental.pallas import tpu as pltpu

_N_STATES = 4
_N_ACTIONS = 2
_HID = 64
_BLOCK_L = 65536


def _mlp_t_kernel(xT_ref, w1_ref, b1_ref, w2_ref, b2_ref, oT_ref):
    h = jnp.maximum(
        jnp.dot(w1_ref[...], xT_ref[...], preferred_element_type=jnp.float32)
        + b1_ref[...],
        0.0,
    )
    oT_ref[...] = (
        jnp.dot(w2_ref[...], h, preferred_element_type=jnp.float32)
        + b2_ref[...]
    )


def kernel(x, w1p, b1p, w2p, b2p):
    B = x.shape[0]
    xT = jnp.transpose(x)
    w1s = w1p[:_HID, :]
    b1s = b1p[:_HID, :]
    w2s = w2p[:_N_ACTIONS, :_HID]
    b2s = b2p[:_N_ACTIONS, :]

    block_l = min(_BLOCK_L, B)
    num_blocks = B // block_l

    oT = pl.pallas_call(
        _mlp_t_kernel,
        out_shape=jax.ShapeDtypeStruct((_N_ACTIONS, B), jnp.float32),
        grid=(num_blocks,),
        in_specs=[
            pl.BlockSpec((_N_STATES, block_l), lambda i: (0, i)),
            pl.BlockSpec((_HID, _N_STATES), lambda i: (0, 0)),
            pl.BlockSpec((_HID, 1), lambda i: (0, 0)),
            pl.BlockSpec((_N_ACTIONS, _HID), lambda i: (0, 0)),
            pl.BlockSpec((_N_ACTIONS, 1), lambda i: (0, 0)),
        ],
        out_specs=pl.BlockSpec((_N_ACTIONS, block_l), lambda i: (0, i)),
        compiler_params=pltpu.CompilerParams(
            dimension_semantics=("parallel",)),
    )(xT, w1s, b1s, w2s, b2s)
    return jnp.transpose(oT)
